# Initial kernel scaffold; baseline (speedup 1.0000x reference)
import functools

import jax
import jax.numpy as jnp
from jax import lax
from jax.experimental import pallas as pl
from jax.experimental.pallas import tpu as pltpu
from jax.experimental.pallas import tpu_sc as plsc

PAD = 1
NC = 2
NS = 16
LANES = 16
NW = NC * NS
G = 32


@functools.partial(jax.jit, static_argnames=("bsz", "seq_len", "dim"))
def _posemb(ids, weights, *, bsz, seq_len, dim):
    n_tokens = bsz * seq_len
    T = n_tokens // NW
    NB = T // G

    mesh = plsc.VectorSubcoreMesh(
        core_axis_name="c", subcore_axis_name="s",
        num_cores=NC, num_subcores=NS)

    @functools.partial(
        pl.kernel,
        out_type=jax.ShapeDtypeStruct((n_tokens, dim), jnp.float32),
        mesh=mesh,
        scratch_types=[
            pltpu.VMEM((seq_len,), jnp.int32),
            pltpu.VMEM((T,), jnp.int32),
            pltpu.VMEM((G, dim), jnp.float32),
            pltpu.VMEM((G, dim), jnp.float32),
            pltpu.SemaphoreType.DMA,
            pltpu.SemaphoreType.DMA,
            pltpu.SemaphoreType.DMA,
            pltpu.SemaphoreType.DMA,
        ],
    )
    def body(ids_hbm, w_hbm, out_hbm, ids_v, pos_v, rows0, rows1,
             gsem0, gsem1, wsem0, wsem1):
        wid = lax.axis_index("s") * NC + lax.axis_index("c")
        start = wid * T
        b = start // seq_len
        c_off = start - b * seq_len

        pltpu.sync_copy(ids_hbm.at[pl.ds(b * seq_len, seq_len)], ids_v)

        def pbody(j, acc):
            v = ids_v[pl.ds(j * LANES, LANES)]
            return acc + jnp.where(v == PAD, 0, 1)

        accv = lax.fori_loop(0, c_off // LANES, pbody,
                             jnp.zeros((LANES,), jnp.int32))
        prefix = jnp.sum(accv)

        def qbody(j, run):
            v = ids_v[pl.ds(c_off + j * LANES, LANES)]
            m = jnp.where(v == PAD, 0, 1)
            cum = plsc.cumsum(m)
            pos_v[pl.ds(j * LANES, LANES)] = (run + cum) * m + PAD
            return run + jnp.sum(m)

        lax.fori_loop(0, T // LANES, qbody, prefix)

        bufs = (rows0, rows1)
        gsems = (gsem0, gsem1)
        wsems = (wsem0, wsem1)
        gathers = [None] * NB
        writes = [None, None]
        for g in range(NB):
            k = g % 2
            if writes[k] is not None:
                writes[k].wait()
                writes[k] = None
            cp = pltpu.make_async_copy(
                w_hbm.at[pos_v.at[pl.ds(g * G, G)]], bufs[k], gsems[k])
            cp.start()
            gathers[g] = cp
            if g > 0:
                pk = (g - 1) % 2
                gathers[g - 1].wait()
                wr = pltpu.make_async_copy(
                    bufs[pk], out_hbm.at[pl.ds(start + (g - 1) * G, G)],
                    wsems[pk])
                wr.start()
                writes[pk] = wr
        lk = (NB - 1) % 2
        gathers[NB - 1].wait()
        wr = pltpu.make_async_copy(
            bufs[lk], out_hbm.at[pl.ds(start + (NB - 1) * G, G)], wsems[lk])
        wr.start()
        writes[lk] = wr
        for k in range(2):
            if writes[k] is not None:
                writes[k].wait()

    return body(ids, weights)


def kernel(input, weights):
    bsz, seq_len = input.shape
    dim = weights.shape[1]
    ids = input.reshape(bsz * seq_len).astype(jnp.int32)
    out = _posemb(ids, weights, bsz=bsz, seq_len=seq_len, dim=dim)
    return out.reshape(bsz, seq_len, dim)


# speedup vs baseline: 1.7127x; 1.7127x over previous
import functools

import jax
import jax.numpy as jnp
from jax import lax
from jax.experimental import pallas as pl
from jax.experimental.pallas import tpu as pltpu
from jax.experimental.pallas import tpu_sc as plsc

PAD = 1
NC = 2
NS = 16
LANES = 16
NW = NC * NS
G = 32


@functools.partial(jax.jit, static_argnames=("bsz", "seq_len", "dim"))
def _posemb(ids, weights, *, bsz, seq_len, dim):
    n_tokens = bsz * seq_len
    T = n_tokens // NW
    NB = T // G

    mesh = plsc.VectorSubcoreMesh(
        core_axis_name="c", subcore_axis_name="s",
        num_cores=NC, num_subcores=NS)

    @functools.partial(
        pl.kernel,
        out_type=jax.ShapeDtypeStruct((bsz, seq_len, dim), jnp.float32),
        mesh=mesh,
        compiler_params=pltpu.CompilerParams(needs_layout_passes=False),
        scratch_types=[
            pltpu.VMEM((seq_len,), jnp.int32),
            pltpu.VMEM((T,), jnp.int32),
            pltpu.VMEM((G, dim), jnp.float32),
            pltpu.VMEM((G, dim), jnp.float32),
            pltpu.SemaphoreType.DMA,
            pltpu.SemaphoreType.DMA,
            pltpu.SemaphoreType.DMA,
            pltpu.SemaphoreType.DMA,
        ],
    )
    def body(ids_hbm, w_hbm, out_hbm, ids_v, pos_v, rows0, rows1,
             gsem0, gsem1, wsem0, wsem1):
        wid = lax.axis_index("s") * NC + lax.axis_index("c")
        start = wid * T
        b = start // seq_len
        c_off = start - b * seq_len

        pltpu.sync_copy(ids_hbm.at[pl.ds(b * seq_len, seq_len)], ids_v)

        def pbody(j, acc):
            v = ids_v[pl.ds(j * LANES, LANES)]
            return acc + jnp.where(v == PAD, 0, 1)

        accv = lax.fori_loop(0, c_off // LANES, pbody,
                             jnp.zeros((LANES,), jnp.int32))
        prefix = jnp.sum(accv)

        def qbody(j, run):
            v = ids_v[pl.ds(c_off + j * LANES, LANES)]
            m = jnp.where(v == PAD, 0, 1)
            cum = plsc.cumsum(m)
            pos_v[pl.ds(j * LANES, LANES)] = (run + cum) * m + PAD
            return run + jnp.sum(m)

        lax.fori_loop(0, T // LANES, qbody, prefix)

        bufs = (rows0, rows1)
        gsems = (gsem0, gsem1)
        wsems = (wsem0, wsem1)
        gathers = [None] * NB
        writes = [None, None]
        for g in range(NB):
            k = g % 2
            if writes[k] is not None:
                writes[k].wait()
                writes[k] = None
            cp = pltpu.make_async_copy(
                w_hbm.at[pos_v.at[pl.ds(g * G, G)]], bufs[k], gsems[k])
            cp.start()
            gathers[g] = cp
            if g > 0:
                pk = (g - 1) % 2
                gathers[g - 1].wait()
                wr = pltpu.make_async_copy(
                    bufs[pk],
                    out_hbm.at[b, pl.ds(c_off + (g - 1) * G, G)],
                    wsems[pk])
                wr.start()
                writes[pk] = wr
        lk = (NB - 1) % 2
        gathers[NB - 1].wait()
        wr = pltpu.make_async_copy(
            bufs[lk], out_hbm.at[b, pl.ds(c_off + (NB - 1) * G, G)], wsems[lk])
        wr.start()
        writes[lk] = wr
        for k in range(2):
            if writes[k] is not None:
                writes[k].wait()
        plsc.subcore_barrier()

    return body(ids, weights)


def kernel(input, weights):
    bsz, seq_len = input.shape
    dim = weights.shape[1]
    ids = input.reshape(bsz * seq_len)
    return _posemb(ids, weights, bsz=bsz, seq_len=seq_len, dim=dim)
